# speedup vs baseline: 3.5282x; 3.5282x over previous
import functools

import jax
import jax.numpy as jnp
from jax import lax
from jax.experimental import pallas as pl
from jax.experimental.pallas import tpu as pltpu
from jax.experimental.pallas import tpu_sc as plsc

_NUM_CORES = 2
_NUM_SUBCORES = 16
_NW = _NUM_CORES * _NUM_SUBCORES

_D = 128
_CHUNK = 80


@functools.partial(jax.jit, static_argnames=("n_edges",))
def _sc_gather(x, src, *, n_edges):
    per_w = n_edges // _NW
    n_chunks = per_w // _CHUNK
    mesh = plsc.VectorSubcoreMesh(core_axis_name="c", subcore_axis_name="s")

    @functools.partial(
        pl.kernel,
        out_type=jax.ShapeDtypeStruct((n_edges, _D), jnp.float32),
        mesh=mesh,
        scratch_types=[
            pltpu.VMEM((per_w,), jnp.int32),
            pltpu.VMEM((_CHUNK, _D), jnp.float32),
            pltpu.SemaphoreType.DMA,
        ],
    )
    def body(x_hbm, src_hbm, out_hbm, idx_v, rows_v, sem):
        wid = lax.axis_index("s") * _NUM_CORES + lax.axis_index("c")
        base = wid * per_w
        pltpu.sync_copy(src_hbm.at[pl.ds(base, per_w)], idx_v)

        def step(j, carry):
            off = j * _CHUNK
            pltpu.async_copy(
                x_hbm.at[idx_v.at[pl.ds(off, _CHUNK)]], rows_v, sem
            ).wait()
            pltpu.sync_copy(rows_v, out_hbm.at[pl.ds(base + off, _CHUNK)])
            return carry

        lax.fori_loop(0, n_chunks, step, 0, unroll=False)

    return body(x, src)


def kernel(x, edge_index):
    src = edge_index[0].astype(jnp.int32)
    return _sc_gather(x, src, n_edges=src.shape[0])


# speedup vs baseline: 5.6491x; 1.6011x over previous
import functools

import jax
import jax.numpy as jnp
from jax import lax
from jax.experimental import pallas as pl
from jax.experimental.pallas import tpu as pltpu
from jax.experimental.pallas import tpu_sc as plsc

_NUM_CORES = 2
_NUM_SUBCORES = 16
_NW = _NUM_CORES * _NUM_SUBCORES

_D = 128
_CHUNK = 80
_NB = 5


@functools.partial(jax.jit, static_argnames=("n_edges",))
def _sc_gather(x, src, *, n_edges):
    per_w = n_edges // _NW
    n_chunks = per_w // _CHUNK
    n_rounds = n_chunks // _NB
    mesh = plsc.VectorSubcoreMesh(core_axis_name="c", subcore_axis_name="s")

    scratch = (
        [pltpu.VMEM((per_w,), jnp.int32)]
        + [pltpu.VMEM((_CHUNK, _D), jnp.float32) for _ in range(_NB)]
        + [pltpu.SemaphoreType.DMA for _ in range(2 * _NB)]
    )

    @functools.partial(
        pl.kernel,
        out_type=jax.ShapeDtypeStruct((n_edges, _D), jnp.float32),
        mesh=mesh,
        scratch_types=scratch,
    )
    def body(x_hbm, src_hbm, out_hbm, idx_v, *bufs_and_sems):
        bufs = bufs_and_sems[:_NB]
        gsem = bufs_and_sems[_NB : 2 * _NB]
        wsem = bufs_and_sems[2 * _NB :]

        wid = lax.axis_index("s") * _NUM_CORES + lax.axis_index("c")
        base = wid * per_w
        pltpu.sync_copy(src_hbm.at[pl.ds(base, per_w)], idx_v)

        def gather_start(c, b):
            pltpu.async_copy(
                x_hbm.at[idx_v.at[pl.ds(c * _CHUNK, _CHUNK)]], bufs[b], gsem[b]
            )

        for b in range(_NB):
            gather_start(b, b)

        def round_body(i, carry):
            c0 = i * _NB
            for b in range(_NB):
                pltpu.make_async_copy(
                    x_hbm.at[pl.ds(0, _CHUNK)], bufs[b], gsem[b]
                ).wait()
                pltpu.async_copy(
                    bufs[b], out_hbm.at[pl.ds(base + (c0 + b) * _CHUNK, _CHUNK)],
                    wsem[b],
                )
            for b in range(_NB):
                pltpu.make_async_copy(
                    bufs[b], out_hbm.at[pl.ds(base, _CHUNK)], wsem[b]
                ).wait()
                nxt = c0 + _NB + b

                @pl.when(nxt < n_chunks)
                def _():
                    gather_start(nxt, b)

            return carry

        lax.fori_loop(0, n_rounds, round_body, 0, unroll=False)

    return body(x, src)


def kernel(x, edge_index):
    src = edge_index[0].astype(jnp.int32)
    return _sc_gather(x, src, n_edges=src.shape[0])


# speedup vs baseline: 8.4165x; 1.4899x over previous
import functools

import jax
import jax.numpy as jnp
from jax import lax
from jax.experimental import pallas as pl
from jax.experimental.pallas import tpu as pltpu
from jax.experimental.pallas import tpu_sc as plsc

_NUM_CORES = 2
_NUM_SUBCORES = 16
_NW = _NUM_CORES * _NUM_SUBCORES

_D = 128
_CHUNK = 40
_NB = 5


@functools.partial(jax.jit, static_argnames=("n_edges",))
def _sc_gather(x, src, *, n_edges):
    per_w = n_edges // _NW
    n_chunks = per_w // _CHUNK
    n_rounds = n_chunks // _NB
    n_nodes = x.shape[0]
    rows_per_s = n_nodes // _NUM_SUBCORES
    mesh = plsc.VectorSubcoreMesh(core_axis_name="c", subcore_axis_name="s")

    scratch = (
        [
            pltpu.VMEM_SHARED((n_nodes, _D), jnp.float32),
            pltpu.VMEM((per_w,), jnp.int32),
        ]
        + [pltpu.VMEM((_CHUNK, _D), jnp.float32) for _ in range(_NB)]
        + [pltpu.SemaphoreType.DMA for _ in range(2 * _NB)]
    )

    @functools.partial(
        pl.kernel,
        out_type=jax.ShapeDtypeStruct((n_edges, _D), jnp.float32),
        mesh=mesh,
        scratch_types=scratch,
    )
    def body(x_hbm, src_hbm, out_hbm, xs, idx_v, *bufs_and_sems):
        bufs = bufs_and_sems[:_NB]
        gsem = bufs_and_sems[_NB : 2 * _NB]
        wsem = bufs_and_sems[2 * _NB :]

        sid = lax.axis_index("s")
        wid = sid * _NUM_CORES + lax.axis_index("c")
        base = wid * per_w
        chunk8 = (n_nodes // _NUM_SUBCORES) & ~7
        tail = n_nodes - chunk8 * _NUM_SUBCORES
        pltpu.sync_copy(
            x_hbm.at[pl.ds(sid * chunk8, chunk8)],
            xs.at[pl.ds(sid * chunk8, chunk8)],
        )
        if tail:

            @pl.when(sid == 0)
            def _():
                pltpu.sync_copy(
                    x_hbm.at[pl.ds(chunk8 * _NUM_SUBCORES, tail)],
                    xs.at[pl.ds(chunk8 * _NUM_SUBCORES, tail)],
                )
        pltpu.sync_copy(src_hbm.at[pl.ds(base, per_w)], idx_v)
        plsc.subcore_barrier()

        def gather_start(c, b):
            pltpu.async_copy(
                xs.at[idx_v.at[pl.ds(c * _CHUNK, _CHUNK)]], bufs[b], gsem[b]
            )

        for b in range(_NB):
            gather_start(b, b)

        def round_body(i, carry):
            c0 = i * _NB
            for b in range(_NB):
                pltpu.make_async_copy(
                    x_hbm.at[pl.ds(0, _CHUNK)], bufs[b], gsem[b]
                ).wait()
                pltpu.async_copy(
                    bufs[b], out_hbm.at[pl.ds(base + (c0 + b) * _CHUNK, _CHUNK)],
                    wsem[b],
                )
            for b in range(_NB):
                pltpu.make_async_copy(
                    bufs[b], out_hbm.at[pl.ds(base, _CHUNK)], wsem[b]
                ).wait()
                nxt = c0 + _NB + b

                @pl.when(nxt < n_chunks)
                def _():
                    gather_start(nxt, b)

            return carry

        lax.fori_loop(0, n_rounds, round_body, 0, unroll=False)

    return body(x, src)


def kernel(x, edge_index):
    src = edge_index[0].astype(jnp.int32)
    return _sc_gather(x, src, n_edges=src.shape[0])
